# Initial kernel scaffold; baseline (speedup 1.0000x reference)
import jax
import jax.numpy as jnp
from jax.experimental import pallas as pl


def kernel(x, edge_index, W0, W1, b0, b1, W_out, b_out):
    raise NotImplementedError("write your pallas kernel here")


# speedup vs baseline: 7.3612x; 7.3612x over previous
import functools

import jax
import jax.numpy as jnp
from jax import lax
from jax.experimental import pallas as pl
from jax.experimental.pallas import tpu as pltpu
from jax.experimental.pallas import tpu_sc as plsc

N = 10000
E = 320000
D = 128
CD = 64
L = 16
NC = 2
NS = 16
NW = NC * NS
EPW = E // NW
BLK = 400
NBLK = EPW // BLK
CH = 80
NCH = BLK // CH
BCH = 16
BNCH = BLK // BCH
ZCH = 16
HN = 10240
SN = 1280
HRT = HN // NS
SRT = SN // NS


def _proj_body(x_ref, w_ref, b_ref, z_ref):
    z = jnp.dot(x_ref[...], w_ref[...], preferred_element_type=jnp.float32)
    z = z + b_ref[...]
    z_ref[...] = jnp.where(z >= 0.0, z, 0.01 * z)


def _proj(x, wc, bc):
    rb = 1000
    return pl.pallas_call(
        _proj_body,
        grid=(N // rb,),
        in_specs=[
            pl.BlockSpec((rb, D), lambda i: (i, 0)),
            pl.BlockSpec((D, D), lambda i: (0, 0)),
            pl.BlockSpec((1, D), lambda i: (0, 0)),
        ],
        out_specs=pl.BlockSpec((rb, D), lambda i: (i, 0)),
        out_shape=jax.ShapeDtypeStruct((N, D), jnp.float32),
    )(x, wc, bc)


def _pass_a_body(z_hbm, src_hbm, dst_hbm, l0_hbm, l1_hbm, pmax_hbm,
                 srcb, dstb, zs, zd, lb0, lb1, mst, sem):
    cid = lax.axis_index("c")
    sid = lax.axis_index("s")
    wid = sid * NC + cid
    base = wid * EPW
    lanes = lax.iota(jnp.int32, L)
    neg = jnp.full((L,), -3e38, dtype=jnp.float32)
    pos = jnp.full((L,), 3e38, dtype=jnp.float32)

    def blk_body(i, carry):
        m0, m1, n0, n1 = carry
        off = base + i * BLK
        for c in range(NCH):
            pltpu.sync_copy(src_hbm.at[pl.ds(off + c * CH, CH)], srcb.at[c])
            pltpu.sync_copy(dst_hbm.at[pl.ds(off + c * CH, CH)], dstb.at[c])
        cps = []
        for c in range(NCH):
            cps.append(pltpu.async_copy(
                z_hbm.at[srcb.at[c]], zs.at[pl.ds(c * CH, CH)], sem))
            cps.append(pltpu.async_copy(
                z_hbm.at[dstb.at[c]], zd.at[pl.ds(c * CH, CH)], sem))
        for cp in cps:
            cp.wait()

        def grp_body(g, carry2):
            m0g, m1g, n0g, n1g = carry2
            acc0 = jnp.zeros((L,), jnp.float32)
            acc1 = jnp.zeros((L,), jnp.float32)
            for j in range(L):
                e = g * L + j
                c0 = zs[e, pl.ds(0, L)] * zd[e, pl.ds(0, L)]
                for k in range(1, 4):
                    c0 = c0 + zs[e, pl.ds(k * L, L)] * zd[e, pl.ds(k * L, L)]
                c1 = zs[e, pl.ds(4 * L, L)] * zd[e, pl.ds(4 * L, L)]
                for k in range(5, 8):
                    c1 = c1 + zs[e, pl.ds(k * L, L)] * zd[e, pl.ds(k * L, L)]
                l0s = jnp.sum(c0)
                l1s = jnp.sum(c1)
                acc0 = jnp.where(lanes == j, l0s, acc0)
                acc1 = jnp.where(lanes == j, l1s, acc1)
            lb0[pl.ds(g * L, L)] = acc0
            lb1[pl.ds(g * L, L)] = acc1
            return (jnp.maximum(m0g, acc0), jnp.maximum(m1g, acc1),
                    jnp.minimum(n0g, acc0), jnp.minimum(n1g, acc1))

        m0, m1, n0, n1 = lax.fori_loop(
            0, BLK // L, grp_body, (m0, m1, n0, n1))
        pltpu.sync_copy(lb0, l0_hbm.at[pl.ds(off, BLK)])
        pltpu.sync_copy(lb1, l1_hbm.at[pl.ds(off, BLK)])
        return m0, m1, n0, n1

    m0, m1, n0, n1 = lax.fori_loop(0, NBLK, blk_body, (neg, neg, pos, pos))
    for idx, v in enumerate((m0, m1, n0, n1)):
        mst[pl.ds(0, L)] = v
        pltpu.sync_copy(mst, pmax_hbm.at[pl.ds(idx * NW * L + wid * L, L)])


def _pass_a(z, src, dst):
    mesh = plsc.VectorSubcoreMesh(
        core_axis_name="c", subcore_axis_name="s", num_cores=NC,
        num_subcores=NS)
    fn = pl.kernel(
        _pass_a_body,
        out_type=(
            jax.ShapeDtypeStruct((E,), jnp.float32),
            jax.ShapeDtypeStruct((E,), jnp.float32),
            jax.ShapeDtypeStruct((4 * NW * L,), jnp.float32),
        ),
        mesh=mesh,
        scratch_types=[
            pltpu.VMEM((NCH, CH), jnp.int32),
            pltpu.VMEM((NCH, CH), jnp.int32),
            pltpu.VMEM((BLK, D), jnp.float32),
            pltpu.VMEM((BLK, D), jnp.float32),
            pltpu.VMEM((BLK,), jnp.float32),
            pltpu.VMEM((BLK,), jnp.float32),
            pltpu.VMEM((L,), jnp.float32),
            pltpu.SemaphoreType.DMA,
        ],
        compiler_params=pltpu.CompilerParams(needs_layout_passes=False),
    )
    return fn(z, src, dst)


def _pass_b_body(z_hbm, src_hbm, dst_hbm, l0_hbm, l1_hbm, pmax_hbm,
                 hacc_hbm, sacc_hbm,
                 srcb, dstb, zs, lb0, lb1, rows, srows,
                 pmv, zbuf, idxb, sidx, h_sh, s_sh, sem):
    cid = lax.axis_index("c")
    sid = lax.axis_index("s")
    wid = sid * NC + cid
    base = wid * EPW
    lanes = lax.iota(jnp.int32, L)

    pltpu.sync_copy(pmax_hbm, pmv)
    mm0 = pmv[pl.ds(0, L)]
    mm1 = pmv[pl.ds(NW * L, L)]
    nn0 = pmv[pl.ds(2 * NW * L, L)]
    nn1 = pmv[pl.ds(3 * NW * L, L)]
    for w in range(1, NW):
        mm0 = jnp.maximum(mm0, pmv[pl.ds(w * L, L)])
        mm1 = jnp.maximum(mm1, pmv[pl.ds((NW + w) * L, L)])
        nn0 = jnp.minimum(nn0, pmv[pl.ds((2 * NW + w) * L, L)])
        nn1 = jnp.minimum(nn1, pmv[pl.ds((3 * NW + w) * L, L)])
    m0 = 0.5 * (jnp.max(mm0) + jnp.min(nn0))
    m1 = 0.5 * (jnp.max(mm1) + jnp.min(nn1))

    zero = jnp.zeros((L,), jnp.float32)
    def zrow(r, _):
        for k in range(D // L):
            zbuf[r, pl.ds(k * L, L)] = zero
        return 0
    lax.fori_loop(0, ZCH, zrow, 0)

    def _mkidx(r0):
        idxb[0, pl.ds(0, L)] = r0 + lanes
        return idxb.at[0]

    def zfill_h(b, _):
        pltpu.sync_copy(zbuf, h_sh.at[_mkidx(sid * HRT + b * ZCH)])
        return 0
    lax.fori_loop(0, HRT // ZCH, zfill_h, 0)
    def zfill_s(b, _):
        pltpu.sync_copy(zbuf, s_sh.at[_mkidx(sid * SRT + b * ZCH)])
        return 0
    lax.fori_loop(0, SRT // ZCH, zfill_s, 0)
    plsc.subcore_barrier()

    def blk_body(i, _):
        off = base + i * BLK
        for c in range(BNCH):
            pltpu.sync_copy(src_hbm.at[pl.ds(off + c * BCH, BCH)], srcb.at[c])
            pltpu.sync_copy(dst_hbm.at[pl.ds(off + c * BCH, BCH)], dstb.at[c])
        pltpu.sync_copy(l0_hbm.at[pl.ds(off, BLK)], lb0)
        pltpu.sync_copy(l1_hbm.at[pl.ds(off, BLK)], lb1)
        for c in range(BNCH):
            pltpu.async_copy(z_hbm.at[srcb.at[c]], zs, sem).wait()
            e0v = jnp.exp(lb0[pl.ds(c * BCH, L)] - m0)
            e1v = jnp.exp(lb1[pl.ds(c * BCH, L)] - m1)
            dstv = dstb[c, pl.ds(0, L)]
            sidx[0, pl.ds(0, L)] = lax.shift_right_logical(dstv, 3)
            dmod = lax.rem(dstv, jnp.full((L,), 8, jnp.int32))
            for j in range(L):
                es0 = e0v[j]
                es1 = e1v[j]
                for k in range(4):
                    rows[j, pl.ds(k * L, L)] = es0 * zs[j, pl.ds(k * L, L)]
                for k in range(4, 8):
                    rows[j, pl.ds(k * L, L)] = es1 * zs[j, pl.ds(k * L, L)]
                sval = jnp.where(lanes == 0, es0,
                                 jnp.where(lanes == 1, es1, 0.0))
                dmj = dmod[j]
                for k in range(8):
                    srows[j, pl.ds(k * L, L)] = jnp.where(
                        dmj == k, sval, zero)
            pltpu.sync_copy(rows, h_sh.at[dstb.at[c]], add=True)
            pltpu.sync_copy(srows, s_sh.at[sidx.at[0]], add=True)
        return 0

    lax.fori_loop(0, NBLK, blk_body, 0)
    plsc.subcore_barrier()
    def out_h(b, _):
        r0 = sid * HRT + b * ZCH
        pltpu.async_copy(h_sh.at[_mkidx(r0)], zbuf, sem).wait()
        pltpu.sync_copy(zbuf, hacc_hbm.at[cid, pl.ds(r0, ZCH)])
        return 0
    lax.fori_loop(0, HRT // ZCH, out_h, 0)
    def out_s(b, _):
        r0 = sid * SRT + b * ZCH
        pltpu.async_copy(s_sh.at[_mkidx(r0)], zbuf, sem).wait()
        pltpu.sync_copy(zbuf, sacc_hbm.at[cid, pl.ds(r0, ZCH)])
        return 0
    lax.fori_loop(0, SRT // ZCH, out_s, 0)


def _pass_b(z, src, dst, l0, l1, pmax):
    mesh = plsc.VectorSubcoreMesh(
        core_axis_name="c", subcore_axis_name="s", num_cores=NC,
        num_subcores=NS)
    fn = pl.kernel(
        _pass_b_body,
        out_type=(
            jax.ShapeDtypeStruct((NC, HN, D), jnp.float32),
            jax.ShapeDtypeStruct((NC, SN, D), jnp.float32),
        ),
        mesh=mesh,
        scratch_types=[
            pltpu.VMEM((BNCH, BCH), jnp.int32),
            pltpu.VMEM((BNCH, BCH), jnp.int32),
            pltpu.VMEM((BCH, D), jnp.float32),
            pltpu.VMEM((BLK,), jnp.float32),
            pltpu.VMEM((BLK,), jnp.float32),
            pltpu.VMEM((BCH, D), jnp.float32),
            pltpu.VMEM((BCH, D), jnp.float32),
            pltpu.VMEM((4 * NW * L,), jnp.float32),
            pltpu.VMEM((ZCH, D), jnp.float32),
            pltpu.VMEM((1, L), jnp.int32),
            pltpu.VMEM((1, L), jnp.int32),
            pltpu.VMEM_SHARED((HN, D), jnp.float32),
            pltpu.VMEM_SHARED((SN, D), jnp.float32),
            pltpu.SemaphoreType.DMA,
        ],
        compiler_params=pltpu.CompilerParams(needs_layout_passes=False),
    )
    return fn(z, src, dst, l0, l1, pmax)


def _final_body(h_ref, s_ref, w_ref, b_ref, o_ref):
    hs = h_ref[0] + h_ref[1]
    ss = s_ref[0] + s_ref[1]
    s0 = ss[:, 0:1]
    s1 = ss[:, 1:2]
    den = jnp.concatenate(
        [jnp.broadcast_to(s0, (s0.shape[0], CD)),
         jnp.broadcast_to(s1, (s1.shape[0], CD))], axis=1)
    h = jnp.where(den > 0.0, hs / jnp.where(den > 0.0, den, 1.0), 0.0)
    o = jnp.dot(h, w_ref[...], preferred_element_type=jnp.float32)
    o = o + b_ref[...]
    o_ref[...] = jnp.where(o >= 0.0, o, 0.01 * o)


def _final(hacc, sacc, w_out, b_out):
    rb = 1000
    return pl.pallas_call(
        _final_body,
        grid=(N // rb,),
        in_specs=[
            pl.BlockSpec((NC, rb, D), lambda i: (0, i, 0)),
            pl.BlockSpec((NC, rb, L), lambda i: (0, i, 0)),
            pl.BlockSpec((D, D), lambda i: (0, 0)),
            pl.BlockSpec((1, D), lambda i: (0, 0)),
        ],
        out_specs=pl.BlockSpec((rb, D), lambda i: (i, 0)),
        out_shape=jax.ShapeDtypeStruct((N, D), jnp.float32),
    )(hacc, sacc, w_out, b_out)


@jax.jit
def kernel(x, edge_index, W0, W1, b0, b1, W_out, b_out):
    src = edge_index[0].astype(jnp.int32)
    dst = edge_index[1].astype(jnp.int32)
    wc = jnp.concatenate([W0, W1], axis=1)
    bc = jnp.concatenate([b0, b1], axis=1)
    z = _proj(x, wc, bc)
    l0, l1, pmax = _pass_a(z, src, dst)
    hacc, sacc = _pass_b(z, src, dst, l0, l1, pmax)
    hn = hacc[:, :N]
    sn = sacc[:, :N // 8].reshape(NC, N, L)
    return _final(hn, sn, W_out, b_out)
